# baseline (speedup 1.0000x reference)
import functools

import jax
import jax.numpy as jnp
from jax import lax
from jax.experimental import pallas as pl
from jax.experimental.pallas import tpu as pltpu
from jax.experimental.pallas import tpu_sc as plsc

B = 4
N = 4096
S = 512
K = 32
CIN = 64
NQ = B * S
NROWS = B * S * K
DPAD = 16
DTAB = 128
EPS = 1e-5


def _fps_body(x_ref, y_ref, z_ref, nx_ref, ny_ref, nz_ref):
    X = x_ref[...]
    Y = y_ref[...]
    Z = z_ref[...]
    lanes = lax.broadcasted_iota(jnp.int32, (B, N), 1)
    cols = lax.broadcasted_iota(jnp.int32, (B, S), 1)

    def body(i, carry):
        dist, far, ax, ay, az = carry
        onehot = lanes == far
        cx = jnp.sum(jnp.where(onehot, X, 0.0), axis=1, keepdims=True)
        cy = jnp.sum(jnp.where(onehot, Y, 0.0), axis=1, keepdims=True)
        cz = jnp.sum(jnp.where(onehot, Z, 0.0), axis=1, keepdims=True)
        sel = cols == i
        ax = jnp.where(sel, cx, ax)
        ay = jnp.where(sel, cy, ay)
        az = jnp.where(sel, cz, az)
        dx = X - cx
        dy = Y - cy
        dz = Z - cz
        d = (dx * dx + dy * dy) + dz * dz
        dist = jnp.minimum(dist, d)
        m = jnp.max(dist, axis=1, keepdims=True)
        far = jnp.min(jnp.where(dist == m, lanes, N), axis=1, keepdims=True)
        return (dist, far, ax, ay, az)

    init = (
        jnp.full((B, N), jnp.inf, jnp.float32),
        jnp.zeros((B, 1), jnp.int32),
        jnp.zeros((B, S), jnp.float32),
        jnp.zeros((B, S), jnp.float32),
        jnp.zeros((B, S), jnp.float32),
    )
    _, _, ax, ay, az = lax.fori_loop(0, S, body, init)
    nx_ref[...] = ax
    ny_ref[...] = ay
    nz_ref[...] = az


def _fps(x, y, z):
    out = jax.ShapeDtypeStruct((B, S), jnp.float32)
    return pl.pallas_call(
        _fps_body,
        out_shape=(out, out, out),
    )(x, y, z)


def _knn_body(src_ref, dstT_ref, idx_ref):
    b = pl.program_id(0)
    src = src_ref[...]
    dstT = dstT_ref[...]
    cross = jnp.dot(src, dstT, preferred_element_type=jnp.float32)
    s2 = jnp.sum(src * src, axis=1, keepdims=True)
    d2 = jnp.sum(dstT * dstT, axis=0, keepdims=True)
    dist = s2 - 2.0 * cross + d2
    kcols = lax.broadcasted_iota(jnp.int32, (S, K), 1)
    base = b * N
    P = 4
    W = N // P
    lanesW = lax.broadcasted_iota(jnp.int32, (S, W), 1)
    parts = [dist[:, p * W:(p + 1) * W] for p in range(P)]
    accd = [jnp.full((S, K), jnp.inf, jnp.float32) for _ in range(P)]
    acci = [jnp.zeros((S, K), jnp.int32) for _ in range(P)]
    for k in range(K):
        for p in range(P):
            m = jnp.min(parts[p], axis=1, keepdims=True)
            cand = jnp.where(parts[p] == m, lanesW, W)
            sel = jnp.min(cand, axis=1, keepdims=True)
            accd[p] = jnp.where(kcols == k, m, accd[p])
            acci[p] = jnp.where(kcols == k, sel + p * W, acci[p])
            parts[p] = jnp.where(lanesW == sel, jnp.inf, parts[p])
    vals = jnp.concatenate(accd, axis=1)
    idxs = jnp.concatenate(acci, axis=1)
    acc = jnp.zeros((S, K), jnp.int32)
    for k in range(K):
        m = jnp.min(vals, axis=1, keepdims=True)
        sel = jnp.min(jnp.where(vals == m, idxs, N), axis=1, keepdims=True)
        acc = jnp.where(kcols == k, sel + base, acc)
        vals = jnp.where(idxs == sel, jnp.inf, vals)
    idx_ref[...] = acc


def _knn(src_pad, dstT_pad):
    return pl.pallas_call(
        _knn_body,
        grid=(B,),
        in_specs=[
            pl.BlockSpec((S, 128), lambda b: (b, 0)),
            pl.BlockSpec((128, N), lambda b: (b, 0)),
        ],
        out_specs=pl.BlockSpec((S, K), lambda b: (b, 0)),
        out_shape=jax.ShapeDtypeStruct((NQ, K), jnp.int32),
    )(src_pad, dstT_pad)


def _gather_rows(table, idx2d):
    mesh = plsc.VectorSubcoreMesh(core_axis_name="c", subcore_axis_name="s")

    @functools.partial(
        pl.kernel,
        mesh=mesh,
        out_type=jax.ShapeDtypeStruct((NROWS, DTAB), jnp.float32),
        scratch_types=[
            pltpu.VMEM((4, 128), jnp.int32),
            pltpu.VMEM((512, DTAB), jnp.float32),
            pltpu.SemaphoreType.DMA,
        ],
    )
    def k(table_hbm, idx_hbm, out_hbm, idx_v, rows_v, sem):
        wid = lax.axis_index("s") * 2 + lax.axis_index("c")
        for c in range(4):
            chunk = wid * 4 + c
            pltpu.sync_copy(idx_hbm.at[pl.ds(chunk * 4, 4)], idx_v)
            copies = []
            for j in range(4):
                copies.append(
                    pltpu.async_copy(
                        table_hbm.at[idx_v.at[j]],
                        rows_v.at[pl.ds(j * 128, 128)],
                        sem,
                    )
                )
            for cp in copies:
                cp.wait()
            pltpu.sync_copy(rows_v, out_hbm.at[pl.ds(chunk * 512, 512)])

    return k(table, idx2d)


def _m1_body(g_ref, wa_ref, nx_ref, wx_ref, y_ref, st_ref):
    t = pl.program_id(0)
    y = jnp.dot(g_ref[...], wa_ref[...], preferred_element_type=jnp.float32)
    offs = jnp.dot(nx_ref[...], wx_ref[...], preferred_element_type=jnp.float32)
    y = y - offs
    y_ref[...] = y
    s = jnp.sum(y, axis=0, keepdims=True)
    ss = jnp.sum(y * y, axis=0, keepdims=True)
    st = jnp.concatenate([s, ss, jnp.zeros((6, y.shape[1]), jnp.float32)], axis=0)

    @pl.when(t == 0)
    def _():
        st_ref[...] = st

    @pl.when(t > 0)
    def _():
        st_ref[...] += st


def _m1(g, wa, nx_pad, wx_pad):
    return pl.pallas_call(
        _m1_body,
        grid=(K,),
        in_specs=[
            pl.BlockSpec((NQ, DTAB), lambda t: (t, 0)),
            pl.BlockSpec((DTAB, 128), lambda t: (0, 0)),
            pl.BlockSpec((NQ, 128), lambda t: (0, 0)),
            pl.BlockSpec((128, 128), lambda t: (0, 0)),
        ],
        out_specs=[
            pl.BlockSpec((NQ, 128), lambda t: (t, 0)),
            pl.BlockSpec((8, 128), lambda t: (0, 0)),
        ],
        out_shape=[
            jax.ShapeDtypeStruct((NROWS, 128), jnp.float32),
            jax.ShapeDtypeStruct((8, 128), jnp.float32),
        ],
    )(g, wa, nx_pad, wx_pad)


def _m2_body(y_ref, sc_ref, w_ref, o_ref, st_ref):
    t = pl.program_id(0)
    scale = sc_ref[0:1, :]
    shift = sc_ref[1:2, :]
    a = jnp.maximum(y_ref[...] * scale + shift, 0.0)
    y = jnp.dot(a, w_ref[...], preferred_element_type=jnp.float32)
    o_ref[...] = y
    s = jnp.sum(y, axis=0, keepdims=True)
    ss = jnp.sum(y * y, axis=0, keepdims=True)
    st = jnp.concatenate([s, ss, jnp.zeros((6, y.shape[1]), jnp.float32)], axis=0)

    @pl.when(t == 0)
    def _():
        st_ref[...] = st

    @pl.when(t > 0)
    def _():
        st_ref[...] += st


def _m2(y1, sc1, w2t):
    return pl.pallas_call(
        _m2_body,
        grid=(K,),
        in_specs=[
            pl.BlockSpec((NQ, 128), lambda t: (t, 0)),
            pl.BlockSpec((8, 128), lambda t: (0, 0)),
            pl.BlockSpec((128, 128), lambda t: (0, 0)),
        ],
        out_specs=[
            pl.BlockSpec((NQ, 128), lambda t: (t, 0)),
            pl.BlockSpec((8, 128), lambda t: (0, 0)),
        ],
        out_shape=[
            jax.ShapeDtypeStruct((NROWS, 128), jnp.float32),
            jax.ShapeDtypeStruct((8, 128), jnp.float32),
        ],
    )(y1, sc1, w2t)


def _m3_body(y_ref, sc_ref, w_ref, mx_ref, st_ref):
    t = pl.program_id(0)
    scale = sc_ref[0:1, :]
    shift = sc_ref[1:2, :]
    a = jnp.maximum(y_ref[...] * scale + shift, 0.0)
    y = jnp.dot(a, w_ref[...], preferred_element_type=jnp.float32)
    s = jnp.sum(y, axis=0, keepdims=True)
    ss = jnp.sum(y * y, axis=0, keepdims=True)
    st = jnp.concatenate([s, ss, jnp.zeros((6, y.shape[1]), jnp.float32)], axis=0)

    @pl.when(t == 0)
    def _():
        mx_ref[...] = y
        st_ref[...] = st

    @pl.when(t > 0)
    def _():
        mx_ref[...] = jnp.maximum(mx_ref[...], y)
        st_ref[...] += st


def _m3(y2, sc2, w3t):
    return pl.pallas_call(
        _m3_body,
        grid=(K,),
        in_specs=[
            pl.BlockSpec((NQ, 128), lambda t: (t, 0)),
            pl.BlockSpec((8, 128), lambda t: (0, 0)),
            pl.BlockSpec((128, 256), lambda t: (0, 0)),
        ],
        out_specs=[
            pl.BlockSpec((NQ, 256), lambda t: (0, 0)),
            pl.BlockSpec((8, 256), lambda t: (0, 0)),
        ],
        out_shape=[
            jax.ShapeDtypeStruct((NQ, 256), jnp.float32),
            jax.ShapeDtypeStruct((8, 256), jnp.float32),
        ],
    )(y2, sc2, w3t)


def _m4_body(mx_ref, sc_ref, o_ref):
    scale = sc_ref[0:1, :]
    shift = sc_ref[1:2, :]
    o_ref[...] = jnp.maximum(mx_ref[...] * scale + shift, 0.0)


def _m4(mx, sc3):
    return pl.pallas_call(
        _m4_body,
        out_shape=jax.ShapeDtypeStruct((NQ, 256), jnp.float32),
    )(mx, sc3)


def _bn_scale_shift(st, g, b, c):
    mean = st[0] / float(NROWS)
    var = st[1] / float(NROWS) - mean * mean
    rstd = g * lax.rsqrt(var + EPS)
    sc = jnp.zeros((8, c), jnp.float32)
    sc = sc.at[0].set(rstd)
    sc = sc.at[1].set(b - mean * rstd)
    return sc


def kernel(xyz, points, W1, g1, b1, W2, g2, b2, W3, g3, b3):
    x = xyz[:, :, 0]
    y = xyz[:, :, 1]
    z = xyz[:, :, 2]

    nx, ny, nz = _fps(x, y, z)
    new_xyz = jnp.stack([nx, ny, nz], axis=-1)

    src_pad = jnp.zeros((NQ, 128), jnp.float32)
    src_pad = src_pad.at[:, 0].set(nx.reshape(-1))
    src_pad = src_pad.at[:, 1].set(ny.reshape(-1))
    src_pad = src_pad.at[:, 2].set(nz.reshape(-1))
    dstT_pad = jnp.zeros((B, 128, N), jnp.float32)
    dstT_pad = dstT_pad.at[:, 0, :].set(x)
    dstT_pad = dstT_pad.at[:, 1, :].set(y)
    dstT_pad = dstT_pad.at[:, 2, :].set(z)
    dstT_pad = dstT_pad.reshape(B * 128, N)

    idx = _knn(src_pad, dstT_pad)

    idx_kmaj = jnp.transpose(idx, (1, 0)).reshape(512, 128)

    table = jnp.concatenate(
        [
            jnp.pad(xyz.reshape(B * N, 3), ((0, 0), (0, DPAD - 3))),
            points.reshape(B * N, CIN),
            jnp.zeros((B * N, DTAB - DPAD - CIN), jnp.float32),
        ],
        axis=1,
    )
    g = _gather_rows(table, idx_kmaj)

    wa = jnp.zeros((DTAB, 128), jnp.float32)
    wa = wa.at[0:3, :].set(W1[:, 0:3].T)
    wa = wa.at[DPAD:DPAD + CIN, :].set(W1[:, 3:].T)
    wx = jnp.zeros((128, 128), jnp.float32)
    wx = wx.at[0:3, :].set(W1[:, 0:3].T)
    nx_pad = src_pad

    y1, st1 = _m1(g, wa, nx_pad, wx)
    sc1 = _bn_scale_shift(st1, g1, b1, 128)
    y2, st2 = _m2(y1, sc1, W2.T)
    sc2 = _bn_scale_shift(st2, g2, b2, 128)
    mx, st3 = _m3(y2, sc2, W3.T)
    sc3 = _bn_scale_shift(st3, g3, b3, 256)
    feat = _m4(mx, sc3).reshape(B, S, 256)

    return (new_xyz, feat)


# speedup vs baseline: 1.1110x; 1.1110x over previous
import functools

import jax
import jax.numpy as jnp
from jax import lax
from jax.experimental import pallas as pl
from jax.experimental.pallas import tpu as pltpu
from jax.experimental.pallas import tpu_sc as plsc

B = 4
N = 4096
S = 512
K = 32
CIN = 64
NQ = B * S
NROWS = B * S * K
DPAD = 16
DTAB = 128
EPS = 1e-5


def _fps_body(x_ref, y_ref, z_ref, nx_ref, ny_ref, nz_ref):
    X = x_ref[...]
    Y = y_ref[...]
    Z = z_ref[...]
    lanes = lax.broadcasted_iota(jnp.int32, (B, N), 1)
    cols = lax.broadcasted_iota(jnp.int32, (B, S), 1)

    def body(i, carry):
        dist, far, ax, ay, az = carry
        onehot = lanes == far
        cx = jnp.sum(jnp.where(onehot, X, 0.0), axis=1, keepdims=True)
        cy = jnp.sum(jnp.where(onehot, Y, 0.0), axis=1, keepdims=True)
        cz = jnp.sum(jnp.where(onehot, Z, 0.0), axis=1, keepdims=True)
        sel = cols == i
        ax = jnp.where(sel, cx, ax)
        ay = jnp.where(sel, cy, ay)
        az = jnp.where(sel, cz, az)
        dx = X - cx
        dy = Y - cy
        dz = Z - cz
        d = (dx * dx + dy * dy) + dz * dz
        dist = jnp.minimum(dist, d)
        m = jnp.max(dist, axis=1, keepdims=True)
        far = jnp.min(jnp.where(dist == m, lanes, N), axis=1, keepdims=True)
        return (dist, far, ax, ay, az)

    init = (
        jnp.full((B, N), jnp.inf, jnp.float32),
        jnp.zeros((B, 1), jnp.int32),
        jnp.zeros((B, S), jnp.float32),
        jnp.zeros((B, S), jnp.float32),
        jnp.zeros((B, S), jnp.float32),
    )
    _, _, ax, ay, az = lax.fori_loop(0, S, body, init)
    nx_ref[...] = ax
    ny_ref[...] = ay
    nz_ref[...] = az


def _fps(x, y, z):
    out = jax.ShapeDtypeStruct((B, S), jnp.float32)
    return pl.pallas_call(
        _fps_body,
        out_shape=(out, out, out),
    )(x, y, z)


def _knn_body(src_ref, dstT_ref, idx_ref):
    b = pl.program_id(0)
    src = src_ref[...]
    dstT = dstT_ref[...]
    cross = jnp.dot(src, dstT, preferred_element_type=jnp.float32)
    s2 = jnp.sum(src * src, axis=1, keepdims=True)
    d2 = jnp.sum(dstT * dstT, axis=0, keepdims=True)
    dist = s2 - 2.0 * cross + d2
    lanes = lax.broadcasted_iota(jnp.int32, (S, N), 1)
    kcols = lax.broadcasted_iota(jnp.int32, (S, K), 1)
    base = b * N
    acc = jnp.zeros((S, K), jnp.int32)
    for k in range(K):
        m = jnp.min(dist, axis=1, keepdims=True)
        hit = dist == m
        sel = jnp.min(jnp.where(hit, lanes, N), axis=1, keepdims=True)
        acc = jnp.where(kcols == k, sel + base, acc)
        dist = jnp.where(hit, jnp.inf, dist)
    idx_ref[...] = acc


def _knn(src_pad, dstT_pad):
    return pl.pallas_call(
        _knn_body,
        grid=(B,),
        in_specs=[
            pl.BlockSpec((S, 128), lambda b: (b, 0)),
            pl.BlockSpec((128, N), lambda b: (b, 0)),
        ],
        out_specs=pl.BlockSpec((S, K), lambda b: (b, 0)),
        out_shape=jax.ShapeDtypeStruct((NQ, K), jnp.int32),
    )(src_pad, dstT_pad)


def _gather_rows(table, idx2d):
    mesh = plsc.VectorSubcoreMesh(core_axis_name="c", subcore_axis_name="s")

    @functools.partial(
        pl.kernel,
        mesh=mesh,
        out_type=jax.ShapeDtypeStruct((NROWS, DTAB), jnp.float32),
        scratch_types=[
            pltpu.VMEM((4, 128), jnp.int32),
            pltpu.VMEM((512, DTAB), jnp.float32),
            pltpu.SemaphoreType.DMA,
        ],
    )
    def k(table_hbm, idx_hbm, out_hbm, idx_v, rows_v, sem):
        wid = lax.axis_index("s") * 2 + lax.axis_index("c")
        for c in range(4):
            chunk = wid * 4 + c
            pltpu.sync_copy(idx_hbm.at[pl.ds(chunk * 4, 4)], idx_v)
            copies = []
            for j in range(4):
                copies.append(
                    pltpu.async_copy(
                        table_hbm.at[idx_v.at[j]],
                        rows_v.at[pl.ds(j * 128, 128)],
                        sem,
                    )
                )
            for cp in copies:
                cp.wait()
            pltpu.sync_copy(rows_v, out_hbm.at[pl.ds(chunk * 512, 512)])

    return k(table, idx2d)


def _m1_body(g_ref, wa_ref, nx_ref, wx_ref, y_ref, st_ref):
    t = pl.program_id(0)
    y = jnp.dot(g_ref[...], wa_ref[...], preferred_element_type=jnp.float32)
    offs = jnp.dot(nx_ref[...], wx_ref[...], preferred_element_type=jnp.float32)
    y = y - offs
    y_ref[...] = y
    s = jnp.sum(y, axis=0, keepdims=True)
    ss = jnp.sum(y * y, axis=0, keepdims=True)
    st = jnp.concatenate([s, ss, jnp.zeros((6, y.shape[1]), jnp.float32)], axis=0)

    @pl.when(t == 0)
    def _():
        st_ref[...] = st

    @pl.when(t > 0)
    def _():
        st_ref[...] += st


def _m1(g, wa, nx_pad, wx_pad):
    return pl.pallas_call(
        _m1_body,
        grid=(K,),
        in_specs=[
            pl.BlockSpec((NQ, DTAB), lambda t: (t, 0)),
            pl.BlockSpec((DTAB, 128), lambda t: (0, 0)),
            pl.BlockSpec((NQ, 128), lambda t: (0, 0)),
            pl.BlockSpec((128, 128), lambda t: (0, 0)),
        ],
        out_specs=[
            pl.BlockSpec((NQ, 128), lambda t: (t, 0)),
            pl.BlockSpec((8, 128), lambda t: (0, 0)),
        ],
        out_shape=[
            jax.ShapeDtypeStruct((NROWS, 128), jnp.float32),
            jax.ShapeDtypeStruct((8, 128), jnp.float32),
        ],
    )(g, wa, nx_pad, wx_pad)


def _m2_body(y_ref, sc_ref, w_ref, o_ref, st_ref):
    t = pl.program_id(0)
    scale = sc_ref[0:1, :]
    shift = sc_ref[1:2, :]
    a = jnp.maximum(y_ref[...] * scale + shift, 0.0)
    y = jnp.dot(a, w_ref[...], preferred_element_type=jnp.float32)
    o_ref[...] = y
    s = jnp.sum(y, axis=0, keepdims=True)
    ss = jnp.sum(y * y, axis=0, keepdims=True)
    st = jnp.concatenate([s, ss, jnp.zeros((6, y.shape[1]), jnp.float32)], axis=0)

    @pl.when(t == 0)
    def _():
        st_ref[...] = st

    @pl.when(t > 0)
    def _():
        st_ref[...] += st


def _m2(y1, sc1, w2t):
    return pl.pallas_call(
        _m2_body,
        grid=(K,),
        in_specs=[
            pl.BlockSpec((NQ, 128), lambda t: (t, 0)),
            pl.BlockSpec((8, 128), lambda t: (0, 0)),
            pl.BlockSpec((128, 128), lambda t: (0, 0)),
        ],
        out_specs=[
            pl.BlockSpec((NQ, 128), lambda t: (t, 0)),
            pl.BlockSpec((8, 128), lambda t: (0, 0)),
        ],
        out_shape=[
            jax.ShapeDtypeStruct((NROWS, 128), jnp.float32),
            jax.ShapeDtypeStruct((8, 128), jnp.float32),
        ],
    )(y1, sc1, w2t)


def _m3_body(y_ref, sc_ref, w_ref, mx_ref, st_ref):
    t = pl.program_id(0)
    scale = sc_ref[0:1, :]
    shift = sc_ref[1:2, :]
    a = jnp.maximum(y_ref[...] * scale + shift, 0.0)
    y = jnp.dot(a, w_ref[...], preferred_element_type=jnp.float32)
    s = jnp.sum(y, axis=0, keepdims=True)
    ss = jnp.sum(y * y, axis=0, keepdims=True)
    st = jnp.concatenate([s, ss, jnp.zeros((6, y.shape[1]), jnp.float32)], axis=0)

    @pl.when(t == 0)
    def _():
        mx_ref[...] = y
        st_ref[...] = st

    @pl.when(t > 0)
    def _():
        mx_ref[...] = jnp.maximum(mx_ref[...], y)
        st_ref[...] += st


def _m3(y2, sc2, w3t):
    return pl.pallas_call(
        _m3_body,
        grid=(K,),
        in_specs=[
            pl.BlockSpec((NQ, 128), lambda t: (t, 0)),
            pl.BlockSpec((8, 128), lambda t: (0, 0)),
            pl.BlockSpec((128, 256), lambda t: (0, 0)),
        ],
        out_specs=[
            pl.BlockSpec((NQ, 256), lambda t: (0, 0)),
            pl.BlockSpec((8, 256), lambda t: (0, 0)),
        ],
        out_shape=[
            jax.ShapeDtypeStruct((NQ, 256), jnp.float32),
            jax.ShapeDtypeStruct((8, 256), jnp.float32),
        ],
    )(y2, sc2, w3t)


def _m4_body(mx_ref, sc_ref, o_ref):
    scale = sc_ref[0:1, :]
    shift = sc_ref[1:2, :]
    o_ref[...] = jnp.maximum(mx_ref[...] * scale + shift, 0.0)


def _m4(mx, sc3):
    return pl.pallas_call(
        _m4_body,
        out_shape=jax.ShapeDtypeStruct((NQ, 256), jnp.float32),
    )(mx, sc3)


def _bn_scale_shift(st, g, b, c):
    mean = st[0] / float(NROWS)
    var = st[1] / float(NROWS) - mean * mean
    rstd = g * lax.rsqrt(var + EPS)
    sc = jnp.zeros((8, c), jnp.float32)
    sc = sc.at[0].set(rstd)
    sc = sc.at[1].set(b - mean * rstd)
    return sc


def kernel(xyz, points, W1, g1, b1, W2, g2, b2, W3, g3, b3):
    x = xyz[:, :, 0]
    y = xyz[:, :, 1]
    z = xyz[:, :, 2]

    nx, ny, nz = _fps(x, y, z)
    new_xyz = jnp.stack([nx, ny, nz], axis=-1)

    src_pad = jnp.zeros((NQ, 128), jnp.float32)
    src_pad = src_pad.at[:, 0].set(nx.reshape(-1))
    src_pad = src_pad.at[:, 1].set(ny.reshape(-1))
    src_pad = src_pad.at[:, 2].set(nz.reshape(-1))
    dstT_pad = jnp.zeros((B, 128, N), jnp.float32)
    dstT_pad = dstT_pad.at[:, 0, :].set(x)
    dstT_pad = dstT_pad.at[:, 1, :].set(y)
    dstT_pad = dstT_pad.at[:, 2, :].set(z)
    dstT_pad = dstT_pad.reshape(B * 128, N)

    idx = _knn(src_pad, dstT_pad)

    idx_kmaj = jnp.transpose(idx, (1, 0)).reshape(512, 128)

    table = jnp.concatenate(
        [
            jnp.pad(xyz.reshape(B * N, 3), ((0, 0), (0, DPAD - 3))),
            points.reshape(B * N, CIN),
            jnp.zeros((B * N, DTAB - DPAD - CIN), jnp.float32),
        ],
        axis=1,
    )
    g = _gather_rows(table, idx_kmaj)

    wa = jnp.zeros((DTAB, 128), jnp.float32)
    wa = wa.at[0:3, :].set(W1[:, 0:3].T)
    wa = wa.at[DPAD:DPAD + CIN, :].set(W1[:, 3:].T)
    wx = jnp.zeros((128, 128), jnp.float32)
    wx = wx.at[0:3, :].set(W1[:, 0:3].T)
    nx_pad = src_pad

    y1, st1 = _m1(g, wa, nx_pad, wx)
    sc1 = _bn_scale_shift(st1, g1, b1, 128)
    y2, st2 = _m2(y1, sc1, W2.T)
    sc2 = _bn_scale_shift(st2, g2, b2, 128)
    mx, st3 = _m3(y2, sc2, W3.T)
    sc3 = _bn_scale_shift(st3, g3, b3, 256)
    feat = _m4(mx, sc3).reshape(B, S, 256)

    return (new_xyz, feat)
